# Initial kernel scaffold; baseline (speedup 1.0000x reference)
import jax
import jax.numpy as jnp
from jax.experimental import pallas as pl


def kernel(node_feats, edge_feats, edge_index, W1, b1, W2, b2, W3, b3, gamma, beta):
    raise NotImplementedError("write your pallas kernel here")


# speedup vs baseline: 1.7315x; 1.7315x over previous
import functools

import jax
import jax.numpy as jnp
from jax import lax
from jax.experimental import pallas as pl
from jax.experimental.pallas import tpu as pltpu
from jax.experimental.pallas import tpu_sc as plsc


def _leaky(x):
    return jnp.where(x > 0, x, 0.01 * x)


def _dot(a, b_bf16):
    return jnp.dot(a.astype(jnp.bfloat16), b_bf16,
                   preferred_element_type=jnp.float32)


_CHUNK = 80


def _sc_gather_body(nchunk, b_per_w, nc, d,
                    p_hbm, src_hbm, dst_hbm, out_hbm,
                    sidx, didx, rows_s, rows_d, sem_s, sem_d):
    wid = lax.axis_index("s") * nc + lax.axis_index("c")
    base = wid * b_per_w

    def chunk_body(i, carry):
        off = pl.multiple_of(base + i * _CHUNK, _CHUNK)
        pltpu.sync_copy(src_hbm.at[pl.ds(off, _CHUNK)], sidx)
        pltpu.sync_copy(dst_hbm.at[pl.ds(off, _CHUNK)], didx)
        cs = pltpu.async_copy(p_hbm.at[sidx], rows_s, sem_s)
        cd = pltpu.async_copy(p_hbm.at[didx], rows_d, sem_d)
        cs.wait()
        cd.wait()

        def add_row(r, c2):
            for j in range(d // 16):
                sl = pl.ds(j * 16, 16)
                rows_s[r, sl] = rows_s[r, sl] + rows_d[r, sl]
            return c2

        lax.fori_loop(0, _CHUNK, add_row, 0, unroll=4)
        pltpu.sync_copy(rows_s, out_hbm.at[pl.ds(off, _CHUNK)])
        return carry

    lax.fori_loop(0, nchunk, chunk_body, 0)


def _sc_gather(p, src, dst):
    n, d = p.shape
    e = src.shape[0]
    info = plsc.get_sparse_core_info()
    nc, ns = info.num_cores, info.num_subcores
    nw = nc * ns
    assert e % (nw * _CHUNK) == 0
    b_per_w = e // nw
    nchunk = b_per_w // _CHUNK
    mesh = plsc.VectorSubcoreMesh(core_axis_name="c", subcore_axis_name="s")
    body = functools.partial(_sc_gather_body, nchunk, b_per_w, nc, d)
    return pl.kernel(
        body,
        out_type=jax.ShapeDtypeStruct((e, d), jnp.float32),
        mesh=mesh,
        scratch_types=[
            pltpu.VMEM((_CHUNK,), jnp.int32),
            pltpu.VMEM((_CHUNK,), jnp.int32),
            pltpu.VMEM((_CHUNK, d), jnp.float32),
            pltpu.VMEM((_CHUNK, d), jnp.float32),
            pltpu.SemaphoreType.DMA,
            pltpu.SemaphoreType.DMA,
        ],
    )(p, src, dst)


def _mlp_body(ef_ref, g_ref, w1e_ref, w1n_ref, b1_ref, w2_ref, b2_ref,
              w3_ref, b3_ref, h_ref, sums_ref, acc_ref):
    i = pl.program_id(0)

    @pl.when(i == 0)
    def _():
        acc_ref[...] = jnp.zeros_like(acc_ref)

    x = (_dot(ef_ref[...], w1e_ref[...]) + _dot(g_ref[...], w1n_ref[...])
         + b1_ref[...])
    h = _leaky(x)
    h = _leaky(_dot(h, w2_ref[...]) + b2_ref[...])
    h = _leaky(_dot(h, w3_ref[...]) + b3_ref[...])
    h_ref[...] = h
    acc_ref[0:1, :] += jnp.sum(h, axis=0, keepdims=True)
    acc_ref[1:2, :] += jnp.sum(h * h, axis=0, keepdims=True)

    @pl.when(i == pl.num_programs(0) - 1)
    def _():
        sums_ref[...] = acc_ref[...]


def _edge_mlp(ef, g, w1e, w1n, b1, w2, b2, w3, b3, block):
    e, de = ef.shape
    dout = g.shape[1]
    grid = e // block
    return pl.pallas_call(
        _mlp_body,
        grid=(grid,),
        in_specs=[
            pl.BlockSpec((block, de), lambda i: (i, 0)),
            pl.BlockSpec((block, dout), lambda i: (i, 0)),
            pl.BlockSpec((de, dout), lambda i: (0, 0)),
            pl.BlockSpec((dout, dout), lambda i: (0, 0)),
            pl.BlockSpec((1, dout), lambda i: (0, 0)),
            pl.BlockSpec((dout, dout), lambda i: (0, 0)),
            pl.BlockSpec((1, dout), lambda i: (0, 0)),
            pl.BlockSpec((dout, dout), lambda i: (0, 0)),
            pl.BlockSpec((1, dout), lambda i: (0, 0)),
        ],
        out_specs=[
            pl.BlockSpec((block, dout), lambda i: (i, 0)),
            pl.BlockSpec((8, dout), lambda i: (0, 0)),
        ],
        out_shape=[
            jax.ShapeDtypeStruct((e, dout), jnp.float32),
            jax.ShapeDtypeStruct((8, dout), jnp.float32),
        ],
        scratch_shapes=[pltpu.VMEM((8, dout), jnp.float32)],
    )(ef, g, w1e, w1n, b1, w2, b2, w3, b3)


def _bn_body(inv_e, sums_ref, gamma_ref, beta_ref, h_ref, o_ref):
    mean = sums_ref[0:1, :] * inv_e
    ex2 = sums_ref[1:2, :] * inv_e
    var = ex2 - mean * mean
    rstd = lax.rsqrt(var + 1e-5)
    scale = gamma_ref[...] * rstd
    shift = beta_ref[...] - mean * scale
    o_ref[...] = h_ref[...] * scale + shift


def _bn_apply(sums, gamma, beta, h, block):
    e, dout = h.shape
    grid = e // block
    return pl.pallas_call(
        functools.partial(_bn_body, 1.0 / e),
        grid=(grid,),
        in_specs=[
            pl.BlockSpec((8, dout), lambda i: (0, 0)),
            pl.BlockSpec((1, dout), lambda i: (0, 0)),
            pl.BlockSpec((1, dout), lambda i: (0, 0)),
            pl.BlockSpec((block, dout), lambda i: (i, 0)),
        ],
        out_specs=pl.BlockSpec((block, dout), lambda i: (i, 0)),
        out_shape=jax.ShapeDtypeStruct((e, dout), jnp.float32),
    )(sums, gamma, beta, h)


def kernel(node_feats, edge_feats, edge_index, W1, b1, W2, b2, W3, b3, gamma, beta):
    de = edge_feats.shape[1]
    dout = W1.shape[1]
    w1e = W1[:de].astype(jnp.bfloat16)
    w1n = W1[de:].astype(jnp.bfloat16)
    src = edge_index[0]
    dst = edge_index[1]

    g = _sc_gather(node_feats, src, dst)
    h, sums = _edge_mlp(edge_feats, g,
                        w1e, w1n, b1.reshape(1, dout),
                        W2.astype(jnp.bfloat16), b2.reshape(1, dout),
                        W3.astype(jnp.bfloat16), b3.reshape(1, dout),
                        block=2000)
    out = _bn_apply(sums, gamma.reshape(1, dout), beta.reshape(1, dout), h,
                    block=4000)
    return out


# speedup vs baseline: 2.3404x; 1.3517x over previous
import functools

import jax
import jax.numpy as jnp
from jax import lax
from jax.experimental import pallas as pl
from jax.experimental.pallas import tpu as pltpu
from jax.experimental.pallas import tpu_sc as plsc


def _leaky(x):
    return jnp.where(x > 0, x, 0.01 * x)


def _dot(a, b_bf16):
    return jnp.dot(a.astype(jnp.bfloat16), b_bf16,
                   preferred_element_type=jnp.float32)


_CHUNK = 80


def _sc_gather_body(nchunk, b_per_w, nc, d,
                    p_hbm, src_hbm, dst_hbm, out_hbm,
                    sidx, didx, rows_s, rows_d, outb, gsem, ssem):
    wid = lax.axis_index("s") * nc + lax.axis_index("c")
    base = wid * b_per_w

    pltpu.sync_copy(src_hbm.at[pl.ds(base, b_per_w)], sidx)
    pltpu.sync_copy(dst_hbm.at[pl.ds(base, b_per_w)], didx)

    def fire_gather(c, b):
        isl = pl.ds(c * _CHUNK, _CHUNK)
        pltpu.async_copy(p_hbm.at[sidx.at[isl]], rows_s[b], gsem[2 * b])
        pltpu.async_copy(p_hbm.at[didx.at[isl]], rows_d[b], gsem[2 * b + 1])

    def wait_gather(b):
        pltpu.make_async_copy(p_hbm.at[sidx.at[pl.ds(0, _CHUNK)]],
                              rows_s[b], gsem[2 * b]).wait()
        pltpu.make_async_copy(p_hbm.at[didx.at[pl.ds(0, _CHUNK)]],
                              rows_d[b], gsem[2 * b + 1]).wait()

    def add(b):
        def add_row(r, c2):
            for j in range(d // 16):
                sl = pl.ds(j * 16, 16)
                outb[b][r, sl] = rows_s[b][r, sl] + rows_d[b][r, sl]
            return c2

        lax.fori_loop(0, _CHUNK, add_row, 0, unroll=4)

    def fire_store(c, b):
        off = pl.multiple_of(base + c * _CHUNK, _CHUNK)
        pltpu.async_copy(outb[b], out_hbm.at[pl.ds(off, _CHUNK)], ssem[b])

    def wait_store(b):
        pltpu.make_async_copy(outb[b], out_hbm.at[pl.ds(0, _CHUNK)],
                              ssem[b]).wait()

    fire_gather(0, 0)
    fire_gather(1, 1)
    wait_gather(0)
    add(0)
    fire_store(0, 0)
    fire_gather(2, 0)
    wait_gather(1)
    add(1)
    fire_store(1, 1)

    def body(j, carry):
        fire_gather(2 * j + 1, 1)
        wait_gather(0)
        wait_store(0)
        add(0)
        fire_store(2 * j, 0)
        fire_gather(2 * j + 2, 0)
        wait_gather(1)
        wait_store(1)
        add(1)
        fire_store(2 * j + 1, 1)
        return carry

    lax.fori_loop(1, nchunk // 2, body, 0)

    wait_gather(0)
    wait_store(0)
    add(0)
    fire_store(nchunk - 1, 0)
    wait_store(0)
    wait_store(1)


def _sc_gather(p, src, dst):
    n, d = p.shape
    e = src.shape[0]
    info = plsc.get_sparse_core_info()
    nc, ns = info.num_cores, info.num_subcores
    nw = nc * ns
    assert e % (nw * _CHUNK) == 0
    b_per_w = e // nw
    nchunk = b_per_w // _CHUNK
    mesh = plsc.VectorSubcoreMesh(core_axis_name="c", subcore_axis_name="s")
    body = functools.partial(_sc_gather_body, nchunk, b_per_w, nc, d)
    return pl.kernel(
        body,
        out_type=jax.ShapeDtypeStruct((e, d), jnp.float32),
        mesh=mesh,
        scratch_types=[
            pltpu.VMEM((b_per_w,), jnp.int32),
            pltpu.VMEM((b_per_w,), jnp.int32),
            [pltpu.VMEM((_CHUNK, d), jnp.float32) for _ in range(2)],
            [pltpu.VMEM((_CHUNK, d), jnp.float32) for _ in range(2)],
            [pltpu.VMEM((_CHUNK, d), jnp.float32) for _ in range(2)],
            [pltpu.SemaphoreType.DMA for _ in range(4)],
            [pltpu.SemaphoreType.DMA for _ in range(2)],
        ],
    )(p, src, dst)


def _mlp_body(ef_ref, g_ref, w1e_ref, w1n_ref, b1_ref, w2_ref, b2_ref,
              w3_ref, b3_ref, h_ref, sums_ref, acc_ref):
    i = pl.program_id(0)

    @pl.when(i == 0)
    def _():
        acc_ref[...] = jnp.zeros_like(acc_ref)

    x = (_dot(ef_ref[...], w1e_ref[...]) + _dot(g_ref[...], w1n_ref[...])
         + b1_ref[...])
    h = _leaky(x)
    h = _leaky(_dot(h, w2_ref[...]) + b2_ref[...])
    h = _leaky(_dot(h, w3_ref[...]) + b3_ref[...])
    h_ref[...] = h
    acc_ref[0:1, :] += jnp.sum(h, axis=0, keepdims=True)
    acc_ref[1:2, :] += jnp.sum(h * h, axis=0, keepdims=True)

    @pl.when(i == pl.num_programs(0) - 1)
    def _():
        sums_ref[...] = acc_ref[...]


def _edge_mlp(ef, g, w1e, w1n, b1, w2, b2, w3, b3, block):
    e, de = ef.shape
    dout = g.shape[1]
    grid = e // block
    return pl.pallas_call(
        _mlp_body,
        grid=(grid,),
        in_specs=[
            pl.BlockSpec((block, de), lambda i: (i, 0)),
            pl.BlockSpec((block, dout), lambda i: (i, 0)),
            pl.BlockSpec((de, dout), lambda i: (0, 0)),
            pl.BlockSpec((dout, dout), lambda i: (0, 0)),
            pl.BlockSpec((1, dout), lambda i: (0, 0)),
            pl.BlockSpec((dout, dout), lambda i: (0, 0)),
            pl.BlockSpec((1, dout), lambda i: (0, 0)),
            pl.BlockSpec((dout, dout), lambda i: (0, 0)),
            pl.BlockSpec((1, dout), lambda i: (0, 0)),
        ],
        out_specs=[
            pl.BlockSpec((block, dout), lambda i: (i, 0)),
            pl.BlockSpec((8, dout), lambda i: (0, 0)),
        ],
        out_shape=[
            jax.ShapeDtypeStruct((e, dout), jnp.float32),
            jax.ShapeDtypeStruct((8, dout), jnp.float32),
        ],
        scratch_shapes=[pltpu.VMEM((8, dout), jnp.float32)],
    )(ef, g, w1e, w1n, b1, w2, b2, w3, b3)


def _bn_body(inv_e, sums_ref, gamma_ref, beta_ref, h_ref, o_ref):
    mean = sums_ref[0:1, :] * inv_e
    ex2 = sums_ref[1:2, :] * inv_e
    var = ex2 - mean * mean
    rstd = lax.rsqrt(var + 1e-5)
    scale = gamma_ref[...] * rstd
    shift = beta_ref[...] - mean * scale
    o_ref[...] = h_ref[...] * scale + shift


def _bn_apply(sums, gamma, beta, h, block):
    e, dout = h.shape
    grid = e // block
    return pl.pallas_call(
        functools.partial(_bn_body, 1.0 / e),
        grid=(grid,),
        in_specs=[
            pl.BlockSpec((8, dout), lambda i: (0, 0)),
            pl.BlockSpec((1, dout), lambda i: (0, 0)),
            pl.BlockSpec((1, dout), lambda i: (0, 0)),
            pl.BlockSpec((block, dout), lambda i: (i, 0)),
        ],
        out_specs=pl.BlockSpec((block, dout), lambda i: (i, 0)),
        out_shape=jax.ShapeDtypeStruct((e, dout), jnp.float32),
    )(sums, gamma, beta, h)


def kernel(node_feats, edge_feats, edge_index, W1, b1, W2, b2, W3, b3, gamma, beta):
    de = edge_feats.shape[1]
    dout = W1.shape[1]
    w1e = W1[:de].astype(jnp.bfloat16)
    w1n = W1[de:].astype(jnp.bfloat16)
    src = edge_index[0]
    dst = edge_index[1]

    g = _sc_gather(node_feats, src, dst)
    h, sums = _edge_mlp(edge_feats, g,
                        w1e, w1n, b1.reshape(1, dout),
                        W2.astype(jnp.bfloat16), b2.reshape(1, dout),
                        W3.astype(jnp.bfloat16), b3.reshape(1, dout),
                        block=2000)
    out = _bn_apply(sums, gamma.reshape(1, dout), beta.reshape(1, dout), h,
                    block=4000)
    return out


# speedup vs baseline: 2.9289x; 1.2515x over previous
import functools

import jax
import jax.numpy as jnp
from jax import lax
from jax.experimental import pallas as pl
from jax.experimental.pallas import tpu as pltpu
from jax.experimental.pallas import tpu_sc as plsc


def _leaky(x):
    return jnp.where(x > 0, x, 0.01 * x)


def _dot(a, b_bf16):
    return jnp.dot(a.astype(jnp.bfloat16), b_bf16,
                   preferred_element_type=jnp.float32)


_CHUNK = 80


def _sc_gather_body(nchunk, b_per_w, nc, d,
                    p_hbm, src_hbm, dst_hbm, out_hbm,
                    sidx, didx, rows_s, rows_d, outb, gsem, ssem):
    wid = lax.axis_index("s") * nc + lax.axis_index("c")
    base = wid * b_per_w

    pltpu.sync_copy(src_hbm.at[pl.ds(base, b_per_w)], sidx)
    pltpu.sync_copy(dst_hbm.at[pl.ds(base, b_per_w)], didx)

    def fire_gather(c, b):
        isl = pl.ds(c * _CHUNK, _CHUNK)
        pltpu.async_copy(p_hbm.at[sidx.at[isl]], rows_s[b], gsem[2 * b])
        pltpu.async_copy(p_hbm.at[didx.at[isl]], rows_d[b], gsem[2 * b + 1])

    def wait_gather(b):
        pltpu.make_async_copy(p_hbm.at[sidx.at[pl.ds(0, _CHUNK)]],
                              rows_s[b], gsem[2 * b]).wait()
        pltpu.make_async_copy(p_hbm.at[didx.at[pl.ds(0, _CHUNK)]],
                              rows_d[b], gsem[2 * b + 1]).wait()

    def add(b):
        def add_row(r, c2):
            for j in range(d // 16):
                sl = pl.ds(j * 16, 16)
                outb[b][r, sl] = rows_s[b][r, sl] + rows_d[b][r, sl]
            return c2

        lax.fori_loop(0, _CHUNK, add_row, 0, unroll=4)

    def fire_store(c, b):
        off = pl.multiple_of(base + c * _CHUNK, _CHUNK)
        pltpu.async_copy(outb[b], out_hbm.at[pl.ds(off, _CHUNK)], ssem[b])

    def wait_store(b):
        pltpu.make_async_copy(outb[b], out_hbm.at[pl.ds(0, _CHUNK)],
                              ssem[b]).wait()

    fire_gather(0, 0)
    fire_gather(1, 1)
    wait_gather(0)
    add(0)
    fire_store(0, 0)
    fire_gather(2, 0)
    wait_gather(1)
    add(1)
    fire_store(1, 1)

    def body(j, carry):
        fire_gather(2 * j + 1, 1)
        wait_gather(0)
        wait_store(0)
        add(0)
        fire_store(2 * j, 0)
        fire_gather(2 * j + 2, 0)
        wait_gather(1)
        wait_store(1)
        add(1)
        fire_store(2 * j + 1, 1)
        return carry

    lax.fori_loop(1, nchunk // 2, body, 0)

    wait_gather(0)
    wait_store(0)
    add(0)
    fire_store(nchunk - 1, 0)
    wait_store(0)
    wait_store(1)


def _sc_gather(p, src, dst):
    n, d = p.shape
    e = src.shape[0]
    info = plsc.get_sparse_core_info()
    nc, ns = info.num_cores, info.num_subcores
    nw = nc * ns
    assert e % (nw * _CHUNK) == 0
    b_per_w = e // nw
    nchunk = b_per_w // _CHUNK
    mesh = plsc.VectorSubcoreMesh(core_axis_name="c", subcore_axis_name="s")
    body = functools.partial(_sc_gather_body, nchunk, b_per_w, nc, d)
    return pl.kernel(
        body,
        out_type=jax.ShapeDtypeStruct((e, d), jnp.float32),
        mesh=mesh,
        scratch_types=[
            pltpu.VMEM((b_per_w,), jnp.int32),
            pltpu.VMEM((b_per_w,), jnp.int32),
            [pltpu.VMEM((_CHUNK, d), jnp.float32) for _ in range(2)],
            [pltpu.VMEM((_CHUNK, d), jnp.float32) for _ in range(2)],
            [pltpu.VMEM((_CHUNK, d), jnp.float32) for _ in range(2)],
            [pltpu.SemaphoreType.DMA for _ in range(4)],
            [pltpu.SemaphoreType.DMA for _ in range(2)],
        ],
    )(p, src, dst)


def _mlp_body(has_prev, *refs):
    if has_prev:
        (ef_ref, g_ref, w1e_ref, w1n_ref, b1_ref, w2_ref, b2_ref,
         w3_ref, b3_ref, _prev_ref, h_ref, sums_ref, acc_ref) = refs
    else:
        (ef_ref, g_ref, w1e_ref, w1n_ref, b1_ref, w2_ref, b2_ref,
         w3_ref, b3_ref, h_ref, sums_ref, acc_ref) = refs
    i = pl.program_id(0)

    @pl.when(i == 0)
    def _():
        acc_ref[...] = jnp.zeros_like(acc_ref)

    x = (_dot(ef_ref[...], w1e_ref[...]) + _dot(g_ref[...], w1n_ref[...])
         + b1_ref[...])
    h = _leaky(x)
    h = _leaky(_dot(h, w2_ref[...]) + b2_ref[...])
    h = _leaky(_dot(h, w3_ref[...]) + b3_ref[...])
    h_ref[...] = h
    acc_ref[0:1, :] += jnp.sum(h, axis=0, keepdims=True)
    acc_ref[1:2, :] += jnp.sum(h * h, axis=0, keepdims=True)

    @pl.when(i == pl.num_programs(0) - 1)
    def _():
        sums_ref[...] = acc_ref[...]


def _edge_mlp_slice(ef_k, g_k, w1e, w1n, b1, w2, b2, w3, b3,
                    e_total, blk_off, block, h_prev=None):
    ek, de = ef_k.shape
    dout = g_k.shape[1]
    grid = ek // block
    in_specs = [
        pl.BlockSpec((block, de), lambda i: (i, 0)),
        pl.BlockSpec((block, dout), lambda i: (i, 0)),
        pl.BlockSpec((de, dout), lambda i: (0, 0)),
        pl.BlockSpec((dout, dout), lambda i: (0, 0)),
        pl.BlockSpec((1, dout), lambda i: (0, 0)),
        pl.BlockSpec((dout, dout), lambda i: (0, 0)),
        pl.BlockSpec((1, dout), lambda i: (0, 0)),
        pl.BlockSpec((dout, dout), lambda i: (0, 0)),
        pl.BlockSpec((1, dout), lambda i: (0, 0)),
    ]
    args = [ef_k, g_k, w1e, w1n, b1, w2, b2, w3, b3]
    kwargs = {}
    if h_prev is not None:
        in_specs.append(pl.BlockSpec(memory_space=pl.ANY))
        args.append(h_prev)
        kwargs["input_output_aliases"] = {9: 0}
    return pl.pallas_call(
        functools.partial(_mlp_body, h_prev is not None),
        grid=(grid,),
        in_specs=in_specs,
        out_specs=[
            pl.BlockSpec((block, dout), lambda i, o=blk_off: (i + o, 0)),
            pl.BlockSpec((8, dout), lambda i: (0, 0)),
        ],
        out_shape=[
            jax.ShapeDtypeStruct((e_total, dout), jnp.float32),
            jax.ShapeDtypeStruct((8, dout), jnp.float32),
        ],
        scratch_shapes=[pltpu.VMEM((8, dout), jnp.float32)],
        **kwargs,
    )(*args)


def _bn_body(inv_e, nslices, sums_ref, gamma_ref, beta_ref, h_ref, o_ref):
    s = sums_ref[0:8, :]
    for k in range(1, nslices):
        s = s + sums_ref[8 * k:8 * (k + 1), :]
    mean = s[0:1, :] * inv_e
    ex2 = s[1:2, :] * inv_e
    var = ex2 - mean * mean
    rstd = lax.rsqrt(var + 1e-5)
    scale = gamma_ref[...] * rstd
    shift = beta_ref[...] - mean * scale
    o_ref[...] = h_ref[...] * scale + shift


def _bn_apply(sums, gamma, beta, h, block):
    e, dout = h.shape
    nslices = sums.shape[0] // 8
    grid = e // block
    return pl.pallas_call(
        functools.partial(_bn_body, 1.0 / e, nslices),
        grid=(grid,),
        in_specs=[
            pl.BlockSpec((8 * nslices, dout), lambda i: (0, 0)),
            pl.BlockSpec((1, dout), lambda i: (0, 0)),
            pl.BlockSpec((1, dout), lambda i: (0, 0)),
            pl.BlockSpec((block, dout), lambda i: (i, 0)),
        ],
        out_specs=pl.BlockSpec((block, dout), lambda i: (i, 0)),
        out_shape=jax.ShapeDtypeStruct((e, dout), jnp.float32),
    )(sums, gamma, beta, h)


_NSLICES = 5
_MLP_BLOCK = 2000


def kernel(node_feats, edge_feats, edge_index, W1, b1, W2, b2, W3, b3, gamma, beta):
    e = edge_feats.shape[0]
    de = edge_feats.shape[1]
    dout = W1.shape[1]
    w1e = W1[:de].astype(jnp.bfloat16)
    w1n = W1[de:].astype(jnp.bfloat16)
    b1r = b1.reshape(1, dout)
    w2 = W2.astype(jnp.bfloat16)
    b2r = b2.reshape(1, dout)
    w3 = W3.astype(jnp.bfloat16)
    b3r = b3.reshape(1, dout)
    src = edge_index[0]
    dst = edge_index[1]

    ek = e // _NSLICES
    gs = [_sc_gather(node_feats, src[k * ek:(k + 1) * ek],
                     dst[k * ek:(k + 1) * ek]) for k in range(_NSLICES)]
    h = None
    sums = []
    for k in range(_NSLICES):
        h, s_k = _edge_mlp_slice(
            edge_feats[k * ek:(k + 1) * ek], gs[k],
            w1e, w1n, b1r, w2, b2r, w3, b3r,
            e_total=e, blk_off=k * (ek // _MLP_BLOCK), block=_MLP_BLOCK,
            h_prev=h)
        sums.append(s_k)
    out = _bn_apply(jnp.concatenate(sums, axis=0),
                    gamma.reshape(1, dout), beta.reshape(1, dout), h,
                    block=4000)
    return out


# speedup vs baseline: 3.0180x; 1.0304x over previous
import functools

import jax
import jax.numpy as jnp
from jax import lax
from jax.experimental import pallas as pl
from jax.experimental.pallas import tpu as pltpu
from jax.experimental.pallas import tpu_sc as plsc


def _leaky(x):
    return jnp.where(x > 0, x, 0.01 * x)


def _dot(a, b_bf16):
    return jnp.dot(a.astype(jnp.bfloat16), b_bf16,
                   preferred_element_type=jnp.float32)


_CHUNK = 80


def _sc_gather_body(nchunk, b_per_w, nc, d,
                    p_hbm, src_hbm, dst_hbm, out_hbm,
                    sidx, didx, rows_s, rows_d, outb, gsem, ssem):
    wid = lax.axis_index("s") * nc + lax.axis_index("c")
    base = wid * b_per_w

    pltpu.sync_copy(src_hbm.at[pl.ds(base, b_per_w)], sidx)
    pltpu.sync_copy(dst_hbm.at[pl.ds(base, b_per_w)], didx)

    def fire_gather(c, b):
        isl = pl.ds(c * _CHUNK, _CHUNK)
        pltpu.async_copy(p_hbm.at[sidx.at[isl]], rows_s[b], gsem[2 * b])
        pltpu.async_copy(p_hbm.at[didx.at[isl]], rows_d[b], gsem[2 * b + 1])

    def wait_gather(b):
        pltpu.make_async_copy(p_hbm.at[sidx.at[pl.ds(0, _CHUNK)]],
                              rows_s[b], gsem[2 * b]).wait()
        pltpu.make_async_copy(p_hbm.at[didx.at[pl.ds(0, _CHUNK)]],
                              rows_d[b], gsem[2 * b + 1]).wait()

    def add(b):
        def add_row(r, c2):
            for j in range(d // 16):
                sl = pl.ds(j * 16, 16)
                outb[b][r, sl] = rows_s[b][r, sl] + rows_d[b][r, sl]
            return c2

        lax.fori_loop(0, _CHUNK, add_row, 0, unroll=4)

    def fire_store(c, b):
        off = pl.multiple_of(base + c * _CHUNK, _CHUNK)
        pltpu.async_copy(outb[b], out_hbm.at[pl.ds(off, _CHUNK)], ssem[b])

    def wait_store(b):
        pltpu.make_async_copy(outb[b], out_hbm.at[pl.ds(0, _CHUNK)],
                              ssem[b]).wait()

    fire_gather(0, 0)
    fire_gather(1, 1)
    wait_gather(0)
    add(0)
    fire_store(0, 0)
    fire_gather(2, 0)
    wait_gather(1)
    add(1)
    fire_store(1, 1)

    def body(j, carry):
        fire_gather(2 * j + 1, 1)
        wait_gather(0)
        wait_store(0)
        add(0)
        fire_store(2 * j, 0)
        fire_gather(2 * j + 2, 0)
        wait_gather(1)
        wait_store(1)
        add(1)
        fire_store(2 * j + 1, 1)
        return carry

    lax.fori_loop(1, nchunk // 2, body, 0)

    wait_gather(0)
    wait_store(0)
    add(0)
    fire_store(nchunk - 1, 0)
    wait_store(0)
    wait_store(1)


def _sc_gather(p, src, dst):
    n, d = p.shape
    e = src.shape[0]
    info = plsc.get_sparse_core_info()
    nc, ns = info.num_cores, info.num_subcores
    nw = nc * ns
    assert e % (nw * _CHUNK) == 0
    b_per_w = e // nw
    nchunk = b_per_w // _CHUNK
    mesh = plsc.VectorSubcoreMesh(core_axis_name="c", subcore_axis_name="s")
    body = functools.partial(_sc_gather_body, nchunk, b_per_w, nc, d)
    return pl.kernel(
        body,
        out_type=jax.ShapeDtypeStruct((e, d), jnp.float32),
        mesh=mesh,
        scratch_types=[
            pltpu.VMEM((b_per_w,), jnp.int32),
            pltpu.VMEM((b_per_w,), jnp.int32),
            [pltpu.VMEM((_CHUNK, d), jnp.float32) for _ in range(2)],
            [pltpu.VMEM((_CHUNK, d), jnp.float32) for _ in range(2)],
            [pltpu.VMEM((_CHUNK, d), jnp.float32) for _ in range(2)],
            [pltpu.SemaphoreType.DMA for _ in range(4)],
            [pltpu.SemaphoreType.DMA for _ in range(2)],
        ],
    )(p, src, dst)


def _mlp_body(has_prev, *refs):
    if has_prev:
        (ef_ref, g_ref, w1e_ref, w1n_ref, b1_ref, w2_ref, b2_ref,
         w3_ref, b3_ref, _prev_ref, h_ref, sums_ref, acc_ref) = refs
    else:
        (ef_ref, g_ref, w1e_ref, w1n_ref, b1_ref, w2_ref, b2_ref,
         w3_ref, b3_ref, h_ref, sums_ref, acc_ref) = refs
    i = pl.program_id(0)

    @pl.when(i == 0)
    def _():
        acc_ref[...] = jnp.zeros_like(acc_ref)

    x = (_dot(ef_ref[...], w1e_ref[...]) + _dot(g_ref[...], w1n_ref[...])
         + b1_ref[...])
    h = _leaky(x)
    h = _leaky(_dot(h, w2_ref[...]) + b2_ref[...])
    h = _leaky(_dot(h, w3_ref[...]) + b3_ref[...])
    h_ref[...] = h.astype(jnp.bfloat16)
    acc_ref[0:1, :] += jnp.sum(h, axis=0, keepdims=True)
    acc_ref[1:2, :] += jnp.sum(h * h, axis=0, keepdims=True)

    @pl.when(i == pl.num_programs(0) - 1)
    def _():
        sums_ref[...] = acc_ref[...]


def _edge_mlp_slice(ef_k, g_k, w1e, w1n, b1, w2, b2, w3, b3,
                    e_total, blk_off, block, h_prev=None):
    ek, de = ef_k.shape
    dout = g_k.shape[1]
    grid = ek // block
    in_specs = [
        pl.BlockSpec((block, de), lambda i: (i, 0)),
        pl.BlockSpec((block, dout), lambda i: (i, 0)),
        pl.BlockSpec((de, dout), lambda i: (0, 0)),
        pl.BlockSpec((dout, dout), lambda i: (0, 0)),
        pl.BlockSpec((1, dout), lambda i: (0, 0)),
        pl.BlockSpec((dout, dout), lambda i: (0, 0)),
        pl.BlockSpec((1, dout), lambda i: (0, 0)),
        pl.BlockSpec((dout, dout), lambda i: (0, 0)),
        pl.BlockSpec((1, dout), lambda i: (0, 0)),
    ]
    args = [ef_k, g_k, w1e, w1n, b1, w2, b2, w3, b3]
    kwargs = {}
    if h_prev is not None:
        in_specs.append(pl.BlockSpec(memory_space=pl.ANY))
        args.append(h_prev)
        kwargs["input_output_aliases"] = {9: 0}
    return pl.pallas_call(
        functools.partial(_mlp_body, h_prev is not None),
        grid=(grid,),
        in_specs=in_specs,
        out_specs=[
            pl.BlockSpec((block, dout), lambda i, o=blk_off: (i + o, 0)),
            pl.BlockSpec((8, dout), lambda i: (0, 0)),
        ],
        out_shape=[
            jax.ShapeDtypeStruct((e_total, dout), jnp.bfloat16),
            jax.ShapeDtypeStruct((8, dout), jnp.float32),
        ],
        scratch_shapes=[pltpu.VMEM((8, dout), jnp.float32)],
        **kwargs,
    )(*args)


def _bn_body(inv_e, nslices, sums_ref, gamma_ref, beta_ref, h_ref, o_ref):
    s = sums_ref[0:8, :]
    for k in range(1, nslices):
        s = s + sums_ref[8 * k:8 * (k + 1), :]
    mean = s[0:1, :] * inv_e
    ex2 = s[1:2, :] * inv_e
    var = ex2 - mean * mean
    rstd = lax.rsqrt(var + 1e-5)
    scale = gamma_ref[...] * rstd
    shift = beta_ref[...] - mean * scale
    o_ref[...] = h_ref[...].astype(jnp.float32) * scale + shift


def _bn_apply(sums, gamma, beta, h, block):
    e, dout = h.shape
    nslices = sums.shape[0] // 8
    grid = e // block
    return pl.pallas_call(
        functools.partial(_bn_body, 1.0 / e, nslices),
        grid=(grid,),
        in_specs=[
            pl.BlockSpec((8 * nslices, dout), lambda i: (0, 0)),
            pl.BlockSpec((1, dout), lambda i: (0, 0)),
            pl.BlockSpec((1, dout), lambda i: (0, 0)),
            pl.BlockSpec((block, dout), lambda i: (i, 0)),
        ],
        out_specs=pl.BlockSpec((block, dout), lambda i: (i, 0)),
        out_shape=jax.ShapeDtypeStruct((e, dout), jnp.float32),
    )(sums, gamma, beta, h)


_NSLICES = 5
_MLP_BLOCK = 2000


def kernel(node_feats, edge_feats, edge_index, W1, b1, W2, b2, W3, b3, gamma, beta):
    e = edge_feats.shape[0]
    de = edge_feats.shape[1]
    dout = W1.shape[1]
    w1e = W1[:de].astype(jnp.bfloat16)
    w1n = W1[de:].astype(jnp.bfloat16)
    b1r = b1.reshape(1, dout)
    w2 = W2.astype(jnp.bfloat16)
    b2r = b2.reshape(1, dout)
    w3 = W3.astype(jnp.bfloat16)
    b3r = b3.reshape(1, dout)
    src = edge_index[0]
    dst = edge_index[1]
    ek = e // _NSLICES
    gs = [_sc_gather(node_feats, src[k * ek:(k + 1) * ek],
                     dst[k * ek:(k + 1) * ek]) for k in range(_NSLICES)]
    h = None
    sums = []
    for k in range(_NSLICES):
        h, s_k = _edge_mlp_slice(
            edge_feats[k * ek:(k + 1) * ek], gs[k],
            w1e, w1n, b1r, w2, b2r, w3, b3r,
            e_total=e, blk_off=k * (ek // _MLP_BLOCK), block=_MLP_BLOCK,
            h_prev=h)
        sums.append(s_k)
    out = _bn_apply(jnp.concatenate(sums, axis=0),
                    gamma.reshape(1, dout), beta.reshape(1, dout), h,
                    block=4000)
    return out


# speedup vs baseline: 3.0267x; 1.0029x over previous
import functools

import jax
import jax.numpy as jnp
from jax import lax
from jax.experimental import pallas as pl
from jax.experimental.pallas import tpu as pltpu
from jax.experimental.pallas import tpu_sc as plsc


def _leaky(x):
    return jnp.where(x > 0, x, 0.01 * x)


def _dot(a, b_bf16):
    return jnp.dot(a.astype(jnp.bfloat16), b_bf16,
                   preferred_element_type=jnp.float32)


_CHUNK = 80


def _sc_gather_body(nchunk, b_per_w, nc, d,
                    p_hbm, src_hbm, dst_hbm, out_hbm,
                    sidx, didx, rows_s, rows_d, outb, gsem, ssem):
    wid = lax.axis_index("s") * nc + lax.axis_index("c")
    base = wid * b_per_w

    pltpu.sync_copy(src_hbm.at[pl.ds(base, b_per_w)], sidx)
    pltpu.sync_copy(dst_hbm.at[pl.ds(base, b_per_w)], didx)

    def fire_gather(c, b):
        isl = pl.ds(c * _CHUNK, _CHUNK)
        pltpu.async_copy(p_hbm.at[sidx.at[isl]], rows_s[b], gsem[2 * b])
        pltpu.async_copy(p_hbm.at[didx.at[isl]], rows_d[b], gsem[2 * b + 1])

    def wait_gather(b):
        pltpu.make_async_copy(p_hbm.at[sidx.at[pl.ds(0, _CHUNK)]],
                              rows_s[b], gsem[2 * b]).wait()
        pltpu.make_async_copy(p_hbm.at[didx.at[pl.ds(0, _CHUNK)]],
                              rows_d[b], gsem[2 * b + 1]).wait()

    def add(b):
        def add_row(r, c2):
            for j in range(d // 16):
                sl = pl.ds(j * 16, 16)
                outb[b][r, sl] = rows_s[b][r, sl] + rows_d[b][r, sl]
            return c2

        lax.fori_loop(0, _CHUNK, add_row, 0, unroll=4)

    def fire_store(c, b):
        off = pl.multiple_of(base + c * _CHUNK, _CHUNK)
        pltpu.async_copy(outb[b], out_hbm.at[pl.ds(off, _CHUNK)], ssem[b])

    def wait_store(b):
        pltpu.make_async_copy(outb[b], out_hbm.at[pl.ds(0, _CHUNK)],
                              ssem[b]).wait()

    fire_gather(0, 0)
    fire_gather(1, 1)
    wait_gather(0)
    add(0)
    fire_store(0, 0)
    fire_gather(2, 0)
    wait_gather(1)
    add(1)
    fire_store(1, 1)

    def body(j, carry):
        fire_gather(2 * j + 1, 1)
        wait_gather(0)
        wait_store(0)
        add(0)
        fire_store(2 * j, 0)
        fire_gather(2 * j + 2, 0)
        wait_gather(1)
        wait_store(1)
        add(1)
        fire_store(2 * j + 1, 1)
        return carry

    lax.fori_loop(1, nchunk // 2, body, 0)

    wait_gather(0)
    wait_store(0)
    add(0)
    fire_store(nchunk - 1, 0)
    wait_store(0)
    wait_store(1)


def _sc_gather(p, src, dst):
    n, d = p.shape
    e = src.shape[0]
    info = plsc.get_sparse_core_info()
    nc, ns = info.num_cores, info.num_subcores
    nw = nc * ns
    assert e % (nw * _CHUNK) == 0
    b_per_w = e // nw
    nchunk = b_per_w // _CHUNK
    mesh = plsc.VectorSubcoreMesh(core_axis_name="c", subcore_axis_name="s")
    body = functools.partial(_sc_gather_body, nchunk, b_per_w, nc, d)
    return pl.kernel(
        body,
        out_type=jax.ShapeDtypeStruct((e, d), jnp.float32),
        mesh=mesh,
        scratch_types=[
            pltpu.VMEM((b_per_w,), jnp.int32),
            pltpu.VMEM((b_per_w,), jnp.int32),
            [pltpu.VMEM((_CHUNK, d), jnp.float32) for _ in range(2)],
            [pltpu.VMEM((_CHUNK, d), jnp.float32) for _ in range(2)],
            [pltpu.VMEM((_CHUNK, d), jnp.float32) for _ in range(2)],
            [pltpu.SemaphoreType.DMA for _ in range(4)],
            [pltpu.SemaphoreType.DMA for _ in range(2)],
        ],
    )(p, src, dst)


def _mlp_body(has_prev, *refs):
    if has_prev:
        (ef_ref, g_ref, w1e_ref, w1n_ref, b1_ref, w2_ref, b2_ref,
         w3_ref, b3_ref, _prev_ref, h_ref, sums_ref, acc_ref) = refs
    else:
        (ef_ref, g_ref, w1e_ref, w1n_ref, b1_ref, w2_ref, b2_ref,
         w3_ref, b3_ref, h_ref, sums_ref, acc_ref) = refs
    i = pl.program_id(0)

    @pl.when(i == 0)
    def _():
        acc_ref[...] = jnp.zeros_like(acc_ref)

    x = (_dot(ef_ref[...], w1e_ref[...]) + _dot(g_ref[...], w1n_ref[...])
         + b1_ref[...])
    h = _leaky(x)
    h = _leaky(_dot(h, w2_ref[...]) + b2_ref[...])
    h = _leaky(_dot(h, w3_ref[...]) + b3_ref[...])
    h_ref[...] = h.astype(jnp.bfloat16)
    acc_ref[0:1, :] += jnp.sum(h, axis=0, keepdims=True)
    acc_ref[1:2, :] += jnp.sum(h * h, axis=0, keepdims=True)

    @pl.when(i == pl.num_programs(0) - 1)
    def _():
        sums_ref[...] = acc_ref[...]


def _edge_mlp_slice(ef_k, g_k, w1e, w1n, b1, w2, b2, w3, b3,
                    e_total, blk_off, block, h_prev=None):
    ek, de = ef_k.shape
    dout = g_k.shape[1]
    grid = ek // block
    in_specs = [
        pl.BlockSpec((block, de), lambda i: (i, 0)),
        pl.BlockSpec((block, dout), lambda i: (i, 0)),
        pl.BlockSpec((de, dout), lambda i: (0, 0)),
        pl.BlockSpec((dout, dout), lambda i: (0, 0)),
        pl.BlockSpec((1, dout), lambda i: (0, 0)),
        pl.BlockSpec((dout, dout), lambda i: (0, 0)),
        pl.BlockSpec((1, dout), lambda i: (0, 0)),
        pl.BlockSpec((dout, dout), lambda i: (0, 0)),
        pl.BlockSpec((1, dout), lambda i: (0, 0)),
    ]
    args = [ef_k, g_k, w1e, w1n, b1, w2, b2, w3, b3]
    kwargs = {}
    if h_prev is not None:
        in_specs.append(pl.BlockSpec(memory_space=pl.ANY))
        args.append(h_prev)
        kwargs["input_output_aliases"] = {9: 0}
    return pl.pallas_call(
        functools.partial(_mlp_body, h_prev is not None),
        grid=(grid,),
        in_specs=in_specs,
        out_specs=[
            pl.BlockSpec((block, dout), lambda i, o=blk_off: (i + o, 0)),
            pl.BlockSpec((8, dout), lambda i: (0, 0)),
        ],
        out_shape=[
            jax.ShapeDtypeStruct((e_total, dout), jnp.bfloat16),
            jax.ShapeDtypeStruct((8, dout), jnp.float32),
        ],
        scratch_shapes=[pltpu.VMEM((8, dout), jnp.float32)],
        **kwargs,
    )(*args)


def _bn_body(inv_e, nslices, sums_ref, gamma_ref, beta_ref, h_ref, o_ref):
    s = sums_ref[0:8, :]
    for k in range(1, nslices):
        s = s + sums_ref[8 * k:8 * (k + 1), :]
    mean = s[0:1, :] * inv_e
    ex2 = s[1:2, :] * inv_e
    var = ex2 - mean * mean
    rstd = lax.rsqrt(var + 1e-5)
    scale = gamma_ref[...] * rstd
    shift = beta_ref[...] - mean * scale
    o_ref[...] = h_ref[...].astype(jnp.float32) * scale + shift


def _bn_apply(sums, gamma, beta, h, block):
    e, dout = h.shape
    nslices = sums.shape[0] // 8
    grid = e // block
    return pl.pallas_call(
        functools.partial(_bn_body, 1.0 / e, nslices),
        grid=(grid,),
        in_specs=[
            pl.BlockSpec((8 * nslices, dout), lambda i: (0, 0)),
            pl.BlockSpec((1, dout), lambda i: (0, 0)),
            pl.BlockSpec((1, dout), lambda i: (0, 0)),
            pl.BlockSpec((block, dout), lambda i: (i, 0)),
        ],
        out_specs=pl.BlockSpec((block, dout), lambda i: (i, 0)),
        out_shape=jax.ShapeDtypeStruct((e, dout), jnp.float32),
    )(sums, gamma, beta, h)


_SLICES = [
    (0, 12800, 1600),
    (12800, 79360, 1280),
    (92160, 79360, 2560),
    (171520, 84480, 2560),
    (256000, 64000, 2000),
]


def kernel(node_feats, edge_feats, edge_index, W1, b1, W2, b2, W3, b3, gamma, beta):
    e = edge_feats.shape[0]
    de = edge_feats.shape[1]
    dout = W1.shape[1]
    w1e = W1[:de].astype(jnp.bfloat16)
    w1n = W1[de:].astype(jnp.bfloat16)
    b1r = b1.reshape(1, dout)
    w2 = W2.astype(jnp.bfloat16)
    b2r = b2.reshape(1, dout)
    w3 = W3.astype(jnp.bfloat16)
    b3r = b3.reshape(1, dout)
    src = edge_index[0]
    dst = edge_index[1]
    assert _SLICES[-1][0] + _SLICES[-1][1] == e
    gs = [_sc_gather(node_feats, src[s:s + n], dst[s:s + n])
          for s, n, _ in _SLICES]
    h = None
    sums = []
    for k, (s, n, blk) in enumerate(_SLICES):
        h, s_k = _edge_mlp_slice(
            edge_feats[s:s + n], gs[k],
            w1e, w1n, b1r, w2, b2r, w3, b3r,
            e_total=e, blk_off=s // blk, block=blk,
            h_prev=h)
        sums.append(s_k)
    out = _bn_apply(jnp.concatenate(sums, axis=0),
                    gamma.reshape(1, dout), beta.reshape(1, dout), h,
                    block=4000)
    return out


# speedup vs baseline: 3.1165x; 1.0297x over previous
import functools

import jax
import jax.numpy as jnp
from jax import lax
from jax.experimental import pallas as pl
from jax.experimental.pallas import tpu as pltpu
from jax.experimental.pallas import tpu_sc as plsc


def _leaky(x):
    return jnp.where(x > 0, x, 0.01 * x)


def _dot(a, b_bf16):
    return jnp.dot(a.astype(jnp.bfloat16), b_bf16,
                   preferred_element_type=jnp.float32)


_CHUNK = 80


def _sc_gather_body(nchunk, b_per_w, nc, d,
                    p_hbm, src_hbm, dst_hbm, out_hbm,
                    sidx, didx, rows_s, rows_d, outb, gsem, ssem):
    wid = lax.axis_index("s") * nc + lax.axis_index("c")
    base = wid * b_per_w

    pltpu.sync_copy(src_hbm.at[pl.ds(base, b_per_w)], sidx)
    pltpu.sync_copy(dst_hbm.at[pl.ds(base, b_per_w)], didx)

    def fire_gather(c, b):
        isl = pl.ds(c * _CHUNK, _CHUNK)
        pltpu.async_copy(p_hbm.at[sidx.at[isl]], rows_s[b], gsem[2 * b])
        pltpu.async_copy(p_hbm.at[didx.at[isl]], rows_d[b], gsem[2 * b + 1])

    def wait_gather(b):
        pltpu.make_async_copy(p_hbm.at[sidx.at[pl.ds(0, _CHUNK)]],
                              rows_s[b], gsem[2 * b]).wait()
        pltpu.make_async_copy(p_hbm.at[didx.at[pl.ds(0, _CHUNK)]],
                              rows_d[b], gsem[2 * b + 1]).wait()

    def add(b):
        def add_row(r, c2):
            for j in range(d // 16):
                sl = pl.ds(j * 16, 16)
                outb[b][r, sl] = rows_s[b][r, sl] + rows_d[b][r, sl]
            return c2

        lax.fori_loop(0, _CHUNK, add_row, 0, unroll=4)

    def fire_store(c, b):
        off = pl.multiple_of(base + c * _CHUNK, _CHUNK)
        pltpu.async_copy(outb[b], out_hbm.at[pl.ds(off, _CHUNK)], ssem[b])

    def wait_store(b):
        pltpu.make_async_copy(outb[b], out_hbm.at[pl.ds(0, _CHUNK)],
                              ssem[b]).wait()

    fire_gather(0, 0)
    fire_gather(1, 1)
    wait_gather(0)
    add(0)
    fire_store(0, 0)
    fire_gather(2, 0)
    wait_gather(1)
    add(1)
    fire_store(1, 1)

    def body(j, carry):
        fire_gather(2 * j + 1, 1)
        wait_gather(0)
        wait_store(0)
        add(0)
        fire_store(2 * j, 0)
        fire_gather(2 * j + 2, 0)
        wait_gather(1)
        wait_store(1)
        add(1)
        fire_store(2 * j + 1, 1)
        return carry

    lax.fori_loop(1, nchunk // 2, body, 0)

    wait_gather(0)
    wait_store(0)
    add(0)
    fire_store(nchunk - 1, 0)
    wait_store(0)
    wait_store(1)


def _sc_gather(p, src, dst):
    n, d = p.shape
    e = src.shape[0]
    info = plsc.get_sparse_core_info()
    nc, ns = info.num_cores, info.num_subcores
    nw = nc * ns
    assert e % (nw * _CHUNK) == 0
    b_per_w = e // nw
    nchunk = b_per_w // _CHUNK
    mesh = plsc.VectorSubcoreMesh(core_axis_name="c", subcore_axis_name="s")
    body = functools.partial(_sc_gather_body, nchunk, b_per_w, nc, d)
    return pl.kernel(
        body,
        out_type=jax.ShapeDtypeStruct((e, d), jnp.float32),
        mesh=mesh,
        scratch_types=[
            pltpu.VMEM((b_per_w,), jnp.int32),
            pltpu.VMEM((b_per_w,), jnp.int32),
            [pltpu.VMEM((_CHUNK, d), jnp.float32) for _ in range(2)],
            [pltpu.VMEM((_CHUNK, d), jnp.float32) for _ in range(2)],
            [pltpu.VMEM((_CHUNK, d), jnp.float32) for _ in range(2)],
            [pltpu.SemaphoreType.DMA for _ in range(4)],
            [pltpu.SemaphoreType.DMA for _ in range(2)],
        ],
    )(p, src, dst)


def _mlp_body(has_prev, *refs):
    if has_prev:
        (ef_ref, g_ref, w1e_ref, w1n_ref, b1_ref, w2_ref, b2_ref,
         w3_ref, b3_ref, _prev_ref, h_ref, sums_ref, acc_ref) = refs
    else:
        (ef_ref, g_ref, w1e_ref, w1n_ref, b1_ref, w2_ref, b2_ref,
         w3_ref, b3_ref, h_ref, sums_ref, acc_ref) = refs
    i = pl.program_id(0)

    @pl.when(i == 0)
    def _():
        acc_ref[...] = jnp.zeros_like(acc_ref)

    x = (_dot(ef_ref[...], w1e_ref[...]) + _dot(g_ref[...], w1n_ref[...])
         + b1_ref[...])
    h = _leaky(x)
    h = _leaky(_dot(h, w2_ref[...]) + b2_ref[...])
    h = _leaky(_dot(h, w3_ref[...]) + b3_ref[...])
    h_ref[...] = h.astype(jnp.bfloat16)
    acc_ref[0:1, :] += jnp.sum(h, axis=0, keepdims=True)
    acc_ref[1:2, :] += jnp.sum(h * h, axis=0, keepdims=True)

    @pl.when(i == pl.num_programs(0) - 1)
    def _():
        sums_ref[...] = acc_ref[...]


def _edge_mlp_slice(ef_k, g_k, w1e, w1n, b1, w2, b2, w3, b3,
                    e_total, blk_off, block, h_prev=None):
    ek, de = ef_k.shape
    dout = g_k.shape[1]
    grid = ek // block
    in_specs = [
        pl.BlockSpec((block, de), lambda i: (i, 0)),
        pl.BlockSpec((block, dout), lambda i: (i, 0)),
        pl.BlockSpec((de, dout), lambda i: (0, 0)),
        pl.BlockSpec((dout, dout), lambda i: (0, 0)),
        pl.BlockSpec((1, dout), lambda i: (0, 0)),
        pl.BlockSpec((dout, dout), lambda i: (0, 0)),
        pl.BlockSpec((1, dout), lambda i: (0, 0)),
        pl.BlockSpec((dout, dout), lambda i: (0, 0)),
        pl.BlockSpec((1, dout), lambda i: (0, 0)),
    ]
    args = [ef_k, g_k, w1e, w1n, b1, w2, b2, w3, b3]
    kwargs = {}
    if h_prev is not None:
        in_specs.append(pl.BlockSpec(memory_space=pl.ANY))
        args.append(h_prev)
        kwargs["input_output_aliases"] = {9: 0}
    return pl.pallas_call(
        functools.partial(_mlp_body, h_prev is not None),
        grid=(grid,),
        in_specs=in_specs,
        out_specs=[
            pl.BlockSpec((block, dout), lambda i, o=blk_off: (i + o, 0)),
            pl.BlockSpec((8, dout), lambda i: (0, 0)),
        ],
        out_shape=[
            jax.ShapeDtypeStruct((e_total, dout), jnp.bfloat16),
            jax.ShapeDtypeStruct((8, dout), jnp.float32),
        ],
        scratch_shapes=[pltpu.VMEM((8, dout), jnp.float32)],
        **kwargs,
    )(*args)


def _bn_body(inv_e, nslices, sums_ref, gamma_ref, beta_ref, h_ref, o_ref):
    s = sums_ref[0:8, :]
    for k in range(1, nslices):
        s = s + sums_ref[8 * k:8 * (k + 1), :]
    mean = s[0:1, :] * inv_e
    ex2 = s[1:2, :] * inv_e
    var = ex2 - mean * mean
    rstd = lax.rsqrt(var + 1e-5)
    scale = gamma_ref[...] * rstd
    shift = beta_ref[...] - mean * scale
    o_ref[...] = h_ref[...].astype(jnp.float32) * scale + shift


def _bn_apply(sums, gamma, beta, h, block):
    e, dout = h.shape
    nslices = sums.shape[0] // 8
    grid = e // block
    return pl.pallas_call(
        functools.partial(_bn_body, 1.0 / e, nslices),
        grid=(grid,),
        in_specs=[
            pl.BlockSpec((8 * nslices, dout), lambda i: (0, 0)),
            pl.BlockSpec((1, dout), lambda i: (0, 0)),
            pl.BlockSpec((1, dout), lambda i: (0, 0)),
            pl.BlockSpec((block, dout), lambda i: (i, 0)),
        ],
        out_specs=pl.BlockSpec((block, dout), lambda i: (i, 0)),
        out_shape=jax.ShapeDtypeStruct((e, dout), jnp.float32),
    )(sums, gamma, beta, h)


_SLICES = [
    (0, 12800, 1600),
    (12800, 79360, 1280),
    (92160, 79360, 2560),
    (171520, 84480, 2560),
    (256000, 64000, 2000),
]


def kernel(node_feats, edge_feats, edge_index, W1, b1, W2, b2, W3, b3, gamma, beta):
    e = edge_feats.shape[0]
    de = edge_feats.shape[1]
    dout = W1.shape[1]
    w1e = W1[:de].astype(jnp.bfloat16)
    w1n = W1[de:].astype(jnp.bfloat16)
    b1r = b1.reshape(1, dout)
    w2 = W2.astype(jnp.bfloat16)
    b2r = b2.reshape(1, dout)
    w3 = W3.astype(jnp.bfloat16)
    b3r = b3.reshape(1, dout)
    src = edge_index[0]
    dst = edge_index[1]
    assert _SLICES[-1][0] + _SLICES[-1][1] == e
    gs = [_sc_gather(node_feats, src[s:s + n], dst[s:s + n])
          for s, n, _ in _SLICES]
    h = None
    sums = []
    for k, (s, n, blk) in enumerate(_SLICES):
        h, s_k = _edge_mlp_slice(
            edge_feats[s:s + n], gs[k],
            w1e, w1n, b1r, w2, b2r, w3, b3r,
            e_total=e, blk_off=s // blk, block=blk,
            h_prev=h)
        sums.append(s_k)
    out = _bn_apply(jnp.concatenate(sums, axis=0),
                    gamma.reshape(1, dout), beta.reshape(1, dout), h,
                    block=8000)
    return out


# speedup vs baseline: 3.2056x; 1.0286x over previous
import functools

import jax
import jax.numpy as jnp
from jax import lax
from jax.experimental import pallas as pl
from jax.experimental.pallas import tpu as pltpu
from jax.experimental.pallas import tpu_sc as plsc


def _leaky(x):
    return jnp.where(x > 0, x, 0.01 * x)


def _dot(a, b_bf16):
    return jnp.dot(a.astype(jnp.bfloat16), b_bf16,
                   preferred_element_type=jnp.float32)


_CHUNK = 80


def _sc_gather_body(nchunk, b_per_w, nc, d,
                    p_hbm, src_hbm, dst_hbm, out_hbm,
                    sidx, didx, rows_s, rows_d, outb, gsem, ssem):
    wid = lax.axis_index("s") * nc + lax.axis_index("c")
    base = wid * b_per_w

    pltpu.sync_copy(src_hbm.at[pl.ds(base, b_per_w)], sidx)
    pltpu.sync_copy(dst_hbm.at[pl.ds(base, b_per_w)], didx)

    def fire_gather(c, b):
        isl = pl.ds(c * _CHUNK, _CHUNK)
        pltpu.async_copy(p_hbm.at[sidx.at[isl]], rows_s[b], gsem[2 * b])
        pltpu.async_copy(p_hbm.at[didx.at[isl]], rows_d[b], gsem[2 * b + 1])

    def wait_gather(b):
        pltpu.make_async_copy(p_hbm.at[sidx.at[pl.ds(0, _CHUNK)]],
                              rows_s[b], gsem[2 * b]).wait()
        pltpu.make_async_copy(p_hbm.at[didx.at[pl.ds(0, _CHUNK)]],
                              rows_d[b], gsem[2 * b + 1]).wait()

    def add(b):
        def add_row(r, c2):
            for j in range(d // 16):
                sl = pl.ds(j * 16, 16)
                outb[b][r, sl] = rows_s[b][r, sl] + rows_d[b][r, sl]
            return c2

        lax.fori_loop(0, _CHUNK, add_row, 0, unroll=4)

    def fire_store(c, b):
        off = pl.multiple_of(base + c * _CHUNK, _CHUNK)
        pltpu.async_copy(outb[b], out_hbm.at[pl.ds(off, _CHUNK)], ssem[b])

    def wait_store(b):
        pltpu.make_async_copy(outb[b], out_hbm.at[pl.ds(0, _CHUNK)],
                              ssem[b]).wait()

    fire_gather(0, 0)
    fire_gather(1, 1)
    wait_gather(0)
    add(0)
    fire_store(0, 0)
    fire_gather(2, 0)
    wait_gather(1)
    add(1)
    fire_store(1, 1)

    def body(j, carry):
        fire_gather(2 * j + 1, 1)
        wait_gather(0)
        wait_store(0)
        add(0)
        fire_store(2 * j, 0)
        fire_gather(2 * j + 2, 0)
        wait_gather(1)
        wait_store(1)
        add(1)
        fire_store(2 * j + 1, 1)
        return carry

    lax.fori_loop(1, nchunk // 2, body, 0)

    wait_gather(0)
    wait_store(0)
    add(0)
    fire_store(nchunk - 1, 0)
    wait_store(0)
    wait_store(1)


def _sc_gather(p, src, dst):
    n, d = p.shape
    e = src.shape[0]
    info = plsc.get_sparse_core_info()
    nc, ns = info.num_cores, info.num_subcores
    nw = nc * ns
    assert e % (nw * _CHUNK) == 0
    b_per_w = e // nw
    nchunk = b_per_w // _CHUNK
    mesh = plsc.VectorSubcoreMesh(core_axis_name="c", subcore_axis_name="s")
    body = functools.partial(_sc_gather_body, nchunk, b_per_w, nc, d)
    return pl.kernel(
        body,
        out_type=jax.ShapeDtypeStruct((e, d), jnp.float32),
        mesh=mesh,
        scratch_types=[
            pltpu.VMEM((b_per_w,), jnp.int32),
            pltpu.VMEM((b_per_w,), jnp.int32),
            [pltpu.VMEM((_CHUNK, d), jnp.float32) for _ in range(2)],
            [pltpu.VMEM((_CHUNK, d), jnp.float32) for _ in range(2)],
            [pltpu.VMEM((_CHUNK, d), jnp.float32) for _ in range(2)],
            [pltpu.SemaphoreType.DMA for _ in range(4)],
            [pltpu.SemaphoreType.DMA for _ in range(2)],
        ],
    )(p, src, dst)


def _mlp_body(has_prev, *refs):
    if has_prev:
        (ef_ref, g_ref, w1e_ref, w1n_ref, b1_ref, w2_ref, b2_ref,
         w3_ref, b3_ref, _prev_ref, h_ref, sums_ref, acc_ref) = refs
    else:
        (ef_ref, g_ref, w1e_ref, w1n_ref, b1_ref, w2_ref, b2_ref,
         w3_ref, b3_ref, h_ref, sums_ref, acc_ref) = refs
    i = pl.program_id(0)

    @pl.when(i == 0)
    def _():
        acc_ref[...] = jnp.zeros_like(acc_ref)

    x = (_dot(ef_ref[...], w1e_ref[...]) + _dot(g_ref[...], w1n_ref[...])
         + b1_ref[...])
    h = _leaky(x)
    h = _leaky(_dot(h, w2_ref[...]) + b2_ref[...])
    h = _leaky(_dot(h, w3_ref[...]) + b3_ref[...])
    h_ref[...] = h.astype(jnp.bfloat16)
    acc_ref[0:1, :] += jnp.sum(h, axis=0, keepdims=True)
    acc_ref[1:2, :] += jnp.sum(h * h, axis=0, keepdims=True)

    @pl.when(i == pl.num_programs(0) - 1)
    def _():
        sums_ref[...] = acc_ref[...]


def _edge_mlp_slice(ef_k, g_k, w1e, w1n, b1, w2, b2, w3, b3,
                    e_total, blk_off, block, h_prev=None):
    ek, de = ef_k.shape
    dout = g_k.shape[1]
    grid = ek // block
    in_specs = [
        pl.BlockSpec((block, de), lambda i: (i, 0)),
        pl.BlockSpec((block, dout), lambda i: (i, 0)),
        pl.BlockSpec((de, dout), lambda i: (0, 0)),
        pl.BlockSpec((dout, dout), lambda i: (0, 0)),
        pl.BlockSpec((1, dout), lambda i: (0, 0)),
        pl.BlockSpec((dout, dout), lambda i: (0, 0)),
        pl.BlockSpec((1, dout), lambda i: (0, 0)),
        pl.BlockSpec((dout, dout), lambda i: (0, 0)),
        pl.BlockSpec((1, dout), lambda i: (0, 0)),
    ]
    args = [ef_k, g_k, w1e, w1n, b1, w2, b2, w3, b3]
    kwargs = {}
    if h_prev is not None:
        in_specs.append(pl.BlockSpec(memory_space=pl.ANY))
        args.append(h_prev)
        kwargs["input_output_aliases"] = {9: 0}
    return pl.pallas_call(
        functools.partial(_mlp_body, h_prev is not None),
        grid=(grid,),
        in_specs=in_specs,
        out_specs=[
            pl.BlockSpec((block, dout), lambda i, o=blk_off: (i + o, 0)),
            pl.BlockSpec((8, dout), lambda i: (0, 0)),
        ],
        out_shape=[
            jax.ShapeDtypeStruct((e_total, dout), jnp.bfloat16),
            jax.ShapeDtypeStruct((8, dout), jnp.float32),
        ],
        scratch_shapes=[pltpu.VMEM((8, dout), jnp.float32)],
        **kwargs,
    )(*args)


def _bn_body(inv_e, nslices, sums_ref, gamma_ref, beta_ref, h_ref, o_ref):
    s = sums_ref[0:8, :]
    for k in range(1, nslices):
        s = s + sums_ref[8 * k:8 * (k + 1), :]
    mean = s[0:1, :] * inv_e
    ex2 = s[1:2, :] * inv_e
    var = ex2 - mean * mean
    rstd = lax.rsqrt(var + 1e-5)
    scale = gamma_ref[...] * rstd
    shift = beta_ref[...] - mean * scale
    o_ref[...] = h_ref[...].astype(jnp.float32) * scale + shift


def _bn_apply(sums, gamma, beta, h, block):
    e, dout = h.shape
    nslices = sums.shape[0] // 8
    grid = e // block
    return pl.pallas_call(
        functools.partial(_bn_body, 1.0 / e, nslices),
        grid=(grid,),
        in_specs=[
            pl.BlockSpec((8 * nslices, dout), lambda i: (0, 0)),
            pl.BlockSpec((1, dout), lambda i: (0, 0)),
            pl.BlockSpec((1, dout), lambda i: (0, 0)),
            pl.BlockSpec((block, dout), lambda i: (i, 0)),
        ],
        out_specs=pl.BlockSpec((block, dout), lambda i: (i, 0)),
        out_shape=jax.ShapeDtypeStruct((e, dout), jnp.float32),
    )(sums, gamma, beta, h)


_SLICES = [
    (0, 12800, 1600),
    (12800, 79360, 1280),
    (92160, 79360, 2560),
    (171520, 84480, 2560),
    (256000, 64000, 4000),
]


def kernel(node_feats, edge_feats, edge_index, W1, b1, W2, b2, W3, b3, gamma, beta):
    e = edge_feats.shape[0]
    de = edge_feats.shape[1]
    dout = W1.shape[1]
    w1e = W1[:de].astype(jnp.bfloat16)
    w1n = W1[de:].astype(jnp.bfloat16)
    b1r = b1.reshape(1, dout)
    w2 = W2.astype(jnp.bfloat16)
    b2r = b2.reshape(1, dout)
    w3 = W3.astype(jnp.bfloat16)
    b3r = b3.reshape(1, dout)
    src = edge_index[0]
    dst = edge_index[1]
    assert _SLICES[-1][0] + _SLICES[-1][1] == e
    gs = [_sc_gather(node_feats, src[s:s + n], dst[s:s + n])
          for s, n, _ in _SLICES]
    h = None
    sums = []
    for k, (s, n, blk) in enumerate(_SLICES):
        h, s_k = _edge_mlp_slice(
            edge_feats[s:s + n], gs[k],
            w1e, w1n, b1r, w2, b2r, w3, b3r,
            e_total=e, blk_off=s // blk, block=blk,
            h_prev=h)
        sums.append(s_k)
    out = _bn_apply(jnp.concatenate(sums, axis=0),
                    gamma.reshape(1, dout), beta.reshape(1, dout), h,
                    block=16000)
    return out


# speedup vs baseline: 3.2212x; 1.0049x over previous
import functools

import jax
import jax.numpy as jnp
from jax import lax
from jax.experimental import pallas as pl
from jax.experimental.pallas import tpu as pltpu
from jax.experimental.pallas import tpu_sc as plsc


def _leaky(x):
    return jnp.where(x > 0, x, 0.01 * x)


def _dot(a, b_bf16):
    return jnp.dot(a.astype(jnp.bfloat16), b_bf16,
                   preferred_element_type=jnp.float32)


_CHUNK = 80


def _sc_gather_body(nchunk, b_per_w, nc, d,
                    p_hbm, src_hbm, dst_hbm, out_hbm,
                    sidx, didx, rows_s, rows_d, outb, gsem, ssem):
    wid = lax.axis_index("s") * nc + lax.axis_index("c")
    base = wid * b_per_w

    pltpu.sync_copy(src_hbm.at[pl.ds(base, b_per_w)], sidx)
    pltpu.sync_copy(dst_hbm.at[pl.ds(base, b_per_w)], didx)

    def fire_gather(c, b):
        isl = pl.ds(c * _CHUNK, _CHUNK)
        pltpu.async_copy(p_hbm.at[sidx.at[isl]], rows_s[b], gsem[2 * b])
        pltpu.async_copy(p_hbm.at[didx.at[isl]], rows_d[b], gsem[2 * b + 1])

    def wait_gather(b):
        pltpu.make_async_copy(p_hbm.at[sidx.at[pl.ds(0, _CHUNK)]],
                              rows_s[b], gsem[2 * b]).wait()
        pltpu.make_async_copy(p_hbm.at[didx.at[pl.ds(0, _CHUNK)]],
                              rows_d[b], gsem[2 * b + 1]).wait()

    def add(b):
        def add_row(r, c2):
            for j in range(d // 16):
                sl = pl.ds(j * 16, 16)
                outb[b][r, sl] = rows_s[b][r, sl] + rows_d[b][r, sl]
            return c2

        lax.fori_loop(0, _CHUNK, add_row, 0, unroll=4)

    def fire_store(c, b):
        off = pl.multiple_of(base + c * _CHUNK, _CHUNK)
        pltpu.async_copy(outb[b], out_hbm.at[pl.ds(off, _CHUNK)], ssem[b])

    def wait_store(b):
        pltpu.make_async_copy(outb[b], out_hbm.at[pl.ds(0, _CHUNK)],
                              ssem[b]).wait()

    fire_gather(0, 0)
    fire_gather(1, 1)
    wait_gather(0)
    add(0)
    fire_store(0, 0)
    fire_gather(2, 0)
    wait_gather(1)
    add(1)
    fire_store(1, 1)

    def body(j, carry):
        fire_gather(2 * j + 1, 1)
        wait_gather(0)
        wait_store(0)
        add(0)
        fire_store(2 * j, 0)
        fire_gather(2 * j + 2, 0)
        wait_gather(1)
        wait_store(1)
        add(1)
        fire_store(2 * j + 1, 1)
        return carry

    lax.fori_loop(1, nchunk // 2, body, 0)

    wait_gather(0)
    wait_store(0)
    add(0)
    fire_store(nchunk - 1, 0)
    wait_store(0)
    wait_store(1)


def _sc_gather(p, src, dst):
    n, d = p.shape
    e = src.shape[0]
    info = plsc.get_sparse_core_info()
    nc, ns = info.num_cores, info.num_subcores
    nw = nc * ns
    assert e % (nw * _CHUNK) == 0
    b_per_w = e // nw
    nchunk = b_per_w // _CHUNK
    mesh = plsc.VectorSubcoreMesh(core_axis_name="c", subcore_axis_name="s")
    body = functools.partial(_sc_gather_body, nchunk, b_per_w, nc, d)
    return pl.kernel(
        body,
        out_type=jax.ShapeDtypeStruct((e, d), jnp.float32),
        mesh=mesh,
        scratch_types=[
            pltpu.VMEM((b_per_w,), jnp.int32),
            pltpu.VMEM((b_per_w,), jnp.int32),
            [pltpu.VMEM((_CHUNK, d), jnp.float32) for _ in range(2)],
            [pltpu.VMEM((_CHUNK, d), jnp.float32) for _ in range(2)],
            [pltpu.VMEM((_CHUNK, d), jnp.float32) for _ in range(2)],
            [pltpu.SemaphoreType.DMA for _ in range(4)],
            [pltpu.SemaphoreType.DMA for _ in range(2)],
        ],
    )(p, src, dst)


def _mlp_body(has_prev, *refs):
    if has_prev:
        (ef_ref, g_ref, w1e_ref, w1n_ref, b1_ref, w2_ref, b2_ref,
         w3_ref, b3_ref, _prev_ref, h_ref, sums_ref, acc_ref) = refs
    else:
        (ef_ref, g_ref, w1e_ref, w1n_ref, b1_ref, w2_ref, b2_ref,
         w3_ref, b3_ref, h_ref, sums_ref, acc_ref) = refs
    i = pl.program_id(0)

    @pl.when(i == 0)
    def _():
        acc_ref[...] = jnp.zeros_like(acc_ref)

    x = (_dot(ef_ref[...], w1e_ref[...]) + _dot(g_ref[...], w1n_ref[...])
         + b1_ref[...])
    h = _leaky(x)
    h = _leaky(_dot(h, w2_ref[...]) + b2_ref[...])
    h = _leaky(_dot(h, w3_ref[...]) + b3_ref[...])
    h_ref[...] = h.astype(jnp.bfloat16)
    acc_ref[0:1, :] += jnp.sum(h, axis=0, keepdims=True)
    acc_ref[1:2, :] += jnp.sum(h * h, axis=0, keepdims=True)

    @pl.when(i == pl.num_programs(0) - 1)
    def _():
        sums_ref[...] = acc_ref[...]


def _edge_mlp_slice(ef_k, g_k, w1e, w1n, b1, w2, b2, w3, b3,
                    e_total, blk_off, block, h_prev=None):
    ek, de = ef_k.shape
    dout = g_k.shape[1]
    grid = ek // block
    in_specs = [
        pl.BlockSpec((block, de), lambda i: (i, 0)),
        pl.BlockSpec((block, dout), lambda i: (i, 0)),
        pl.BlockSpec((de, dout), lambda i: (0, 0)),
        pl.BlockSpec((dout, dout), lambda i: (0, 0)),
        pl.BlockSpec((1, dout), lambda i: (0, 0)),
        pl.BlockSpec((dout, dout), lambda i: (0, 0)),
        pl.BlockSpec((1, dout), lambda i: (0, 0)),
        pl.BlockSpec((dout, dout), lambda i: (0, 0)),
        pl.BlockSpec((1, dout), lambda i: (0, 0)),
    ]
    args = [ef_k, g_k, w1e, w1n, b1, w2, b2, w3, b3]
    kwargs = {}
    if h_prev is not None:
        in_specs.append(pl.BlockSpec(memory_space=pl.ANY))
        args.append(h_prev)
        kwargs["input_output_aliases"] = {9: 0}
    return pl.pallas_call(
        functools.partial(_mlp_body, h_prev is not None),
        grid=(grid,),
        in_specs=in_specs,
        out_specs=[
            pl.BlockSpec((block, dout), lambda i, o=blk_off: (i + o, 0)),
            pl.BlockSpec((8, dout), lambda i: (0, 0)),
        ],
        out_shape=[
            jax.ShapeDtypeStruct((e_total, dout), jnp.bfloat16),
            jax.ShapeDtypeStruct((8, dout), jnp.float32),
        ],
        scratch_shapes=[pltpu.VMEM((8, dout), jnp.float32)],
        **kwargs,
    )(*args)


def _bn_body(inv_e, nslices, sums_ref, gamma_ref, beta_ref, h_ref, o_ref):
    s = sums_ref[0:8, :]
    for k in range(1, nslices):
        s = s + sums_ref[8 * k:8 * (k + 1), :]
    mean = s[0:1, :] * inv_e
    ex2 = s[1:2, :] * inv_e
    var = ex2 - mean * mean
    rstd = lax.rsqrt(var + 1e-5)
    scale = gamma_ref[...] * rstd
    shift = beta_ref[...] - mean * scale
    o_ref[...] = h_ref[...].astype(jnp.float32) * scale + shift


def _bn_apply(sums, gamma, beta, h, block):
    e, dout = h.shape
    nslices = sums.shape[0] // 8
    grid = e // block
    return pl.pallas_call(
        functools.partial(_bn_body, 1.0 / e, nslices),
        grid=(grid,),
        in_specs=[
            pl.BlockSpec((8 * nslices, dout), lambda i: (0, 0)),
            pl.BlockSpec((1, dout), lambda i: (0, 0)),
            pl.BlockSpec((1, dout), lambda i: (0, 0)),
            pl.BlockSpec((block, dout), lambda i: (i, 0)),
        ],
        out_specs=pl.BlockSpec((block, dout), lambda i: (i, 0)),
        out_shape=jax.ShapeDtypeStruct((e, dout), jnp.float32),
    )(sums, gamma, beta, h)


_SLICES = [
    (0, 12800, 1600),
    (12800, 79360, 1280),
    (92160, 79360, 2560),
    (171520, 84480, 2560),
    (256000, 64000, 4000),
]


def kernel(node_feats, edge_feats, edge_index, W1, b1, W2, b2, W3, b3, gamma, beta):
    e = edge_feats.shape[0]
    de = edge_feats.shape[1]
    dout = W1.shape[1]
    w1e = W1[:de].astype(jnp.bfloat16)
    w1n = W1[de:].astype(jnp.bfloat16)
    b1r = b1.reshape(1, dout)
    w2 = W2.astype(jnp.bfloat16)
    b2r = b2.reshape(1, dout)
    w3 = W3.astype(jnp.bfloat16)
    b3r = b3.reshape(1, dout)
    src = edge_index[0]
    dst = edge_index[1]
    assert _SLICES[-1][0] + _SLICES[-1][1] == e
    gs = [_sc_gather(node_feats, src[s:s + n], dst[s:s + n])
          for s, n, _ in _SLICES]
    h = None
    sums = []
    for k, (s, n, blk) in enumerate(_SLICES):
        h, s_k = _edge_mlp_slice(
            edge_feats[s:s + n], gs[k],
            w1e, w1n, b1r, w2, b2r, w3, b3r,
            e_total=e, blk_off=s // blk, block=blk,
            h_prev=h)
        sums.append(s_k)
    out = _bn_apply(jnp.concatenate(sums, axis=0),
                    gamma.reshape(1, dout), beta.reshape(1, dout), h,
                    block=32000)
    return out


# speedup vs baseline: 3.2266x; 1.0017x over previous
import functools

import jax
import jax.numpy as jnp
from jax import lax
from jax.experimental import pallas as pl
from jax.experimental.pallas import tpu as pltpu
from jax.experimental.pallas import tpu_sc as plsc


def _leaky(x):
    return jnp.where(x > 0, x, 0.01 * x)


def _dot(a, b_bf16):
    return jnp.dot(a.astype(jnp.bfloat16), b_bf16,
                   preferred_element_type=jnp.float32)


_CHUNK = 80


def _sc_gather_body(nchunk, b_per_w, nc, d,
                    p_hbm, src_hbm, dst_hbm, out_hbm,
                    sidx, didx, rows_s, rows_d, outb, gsem, ssem):
    wid = lax.axis_index("s") * nc + lax.axis_index("c")
    base = wid * b_per_w

    pltpu.sync_copy(src_hbm.at[pl.ds(base, b_per_w)], sidx)
    pltpu.sync_copy(dst_hbm.at[pl.ds(base, b_per_w)], didx)

    _H = _CHUNK // 2

    def fire_gather(c, b):
        lo = pl.ds(c * _CHUNK, _H)
        hi = pl.ds(c * _CHUNK + _H, _H)
        pltpu.async_copy(p_hbm.at[sidx.at[lo]],
                         rows_s[b].at[pl.ds(0, _H)], gsem[4 * b])
        pltpu.async_copy(p_hbm.at[sidx.at[hi]],
                         rows_s[b].at[pl.ds(_H, _H)], gsem[4 * b + 1])
        pltpu.async_copy(p_hbm.at[didx.at[lo]],
                         rows_d[b].at[pl.ds(0, _H)], gsem[4 * b + 2])
        pltpu.async_copy(p_hbm.at[didx.at[hi]],
                         rows_d[b].at[pl.ds(_H, _H)], gsem[4 * b + 3])

    def wait_gather(b):
        for q in range(4):
            ref = rows_s[b] if q < 2 else rows_d[b]
            pltpu.make_async_copy(p_hbm.at[sidx.at[pl.ds(0, _H)]],
                                  ref.at[pl.ds(0, _H)], gsem[4 * b + q]).wait()

    def add(b):
        def add_row(r, c2):
            for j in range(d // 16):
                sl = pl.ds(j * 16, 16)
                outb[b][r, sl] = rows_s[b][r, sl] + rows_d[b][r, sl]
            return c2

        lax.fori_loop(0, _CHUNK, add_row, 0, unroll=4)

    def fire_store(c, b):
        off = pl.multiple_of(base + c * _CHUNK, _CHUNK)
        pltpu.async_copy(outb[b], out_hbm.at[pl.ds(off, _CHUNK)], ssem[b])

    def wait_store(b):
        pltpu.make_async_copy(outb[b], out_hbm.at[pl.ds(0, _CHUNK)],
                              ssem[b]).wait()

    fire_gather(0, 0)
    fire_gather(1, 1)
    wait_gather(0)
    add(0)
    fire_store(0, 0)
    fire_gather(2, 0)
    wait_gather(1)
    add(1)
    fire_store(1, 1)

    def body(j, carry):
        fire_gather(2 * j + 1, 1)
        wait_gather(0)
        wait_store(0)
        add(0)
        fire_store(2 * j, 0)
        fire_gather(2 * j + 2, 0)
        wait_gather(1)
        wait_store(1)
        add(1)
        fire_store(2 * j + 1, 1)
        return carry

    lax.fori_loop(1, nchunk // 2, body, 0)

    wait_gather(0)
    wait_store(0)
    add(0)
    fire_store(nchunk - 1, 0)
    wait_store(0)
    wait_store(1)


def _sc_gather(p, src, dst):
    n, d = p.shape
    e = src.shape[0]
    info = plsc.get_sparse_core_info()
    nc, ns = info.num_cores, info.num_subcores
    nw = nc * ns
    assert e % (nw * _CHUNK) == 0
    b_per_w = e // nw
    nchunk = b_per_w // _CHUNK
    mesh = plsc.VectorSubcoreMesh(core_axis_name="c", subcore_axis_name="s")
    body = functools.partial(_sc_gather_body, nchunk, b_per_w, nc, d)
    return pl.kernel(
        body,
        out_type=jax.ShapeDtypeStruct((e, d), jnp.float32),
        mesh=mesh,
        scratch_types=[
            pltpu.VMEM((b_per_w,), jnp.int32),
            pltpu.VMEM((b_per_w,), jnp.int32),
            [pltpu.VMEM((_CHUNK, d), jnp.float32) for _ in range(2)],
            [pltpu.VMEM((_CHUNK, d), jnp.float32) for _ in range(2)],
            [pltpu.VMEM((_CHUNK, d), jnp.float32) for _ in range(2)],
            [pltpu.SemaphoreType.DMA for _ in range(8)],
            [pltpu.SemaphoreType.DMA for _ in range(2)],
        ],
    )(p, src, dst)


def _mlp_body(has_prev, *refs):
    if has_prev:
        (ef_ref, g_ref, w1e_ref, w1n_ref, b1_ref, w2_ref, b2_ref,
         w3_ref, b3_ref, _prev_ref, h_ref, sums_ref, acc_ref) = refs
    else:
        (ef_ref, g_ref, w1e_ref, w1n_ref, b1_ref, w2_ref, b2_ref,
         w3_ref, b3_ref, h_ref, sums_ref, acc_ref) = refs
    i = pl.program_id(0)

    @pl.when(i == 0)
    def _():
        acc_ref[...] = jnp.zeros_like(acc_ref)

    x = (_dot(ef_ref[...], w1e_ref[...]) + _dot(g_ref[...], w1n_ref[...])
         + b1_ref[...])
    h = _leaky(x)
    h = _leaky(_dot(h, w2_ref[...]) + b2_ref[...])
    h = _leaky(_dot(h, w3_ref[...]) + b3_ref[...])
    h_ref[...] = h.astype(jnp.bfloat16)
    acc_ref[0:1, :] += jnp.sum(h, axis=0, keepdims=True)
    acc_ref[1:2, :] += jnp.sum(h * h, axis=0, keepdims=True)

    @pl.when(i == pl.num_programs(0) - 1)
    def _():
        sums_ref[...] = acc_ref[...]


def _edge_mlp_slice(ef_k, g_k, w1e, w1n, b1, w2, b2, w3, b3,
                    e_total, blk_off, block, h_prev=None):
    ek, de = ef_k.shape
    dout = g_k.shape[1]
    grid = ek // block
    in_specs = [
        pl.BlockSpec((block, de), lambda i: (i, 0)),
        pl.BlockSpec((block, dout), lambda i: (i, 0)),
        pl.BlockSpec((de, dout), lambda i: (0, 0)),
        pl.BlockSpec((dout, dout), lambda i: (0, 0)),
        pl.BlockSpec((1, dout), lambda i: (0, 0)),
        pl.BlockSpec((dout, dout), lambda i: (0, 0)),
        pl.BlockSpec((1, dout), lambda i: (0, 0)),
        pl.BlockSpec((dout, dout), lambda i: (0, 0)),
        pl.BlockSpec((1, dout), lambda i: (0, 0)),
    ]
    args = [ef_k, g_k, w1e, w1n, b1, w2, b2, w3, b3]
    kwargs = {}
    if h_prev is not None:
        in_specs.append(pl.BlockSpec(memory_space=pl.ANY))
        args.append(h_prev)
        kwargs["input_output_aliases"] = {9: 0}
    return pl.pallas_call(
        functools.partial(_mlp_body, h_prev is not None),
        grid=(grid,),
        in_specs=in_specs,
        out_specs=[
            pl.BlockSpec((block, dout), lambda i, o=blk_off: (i + o, 0)),
            pl.BlockSpec((8, dout), lambda i: (0, 0)),
        ],
        out_shape=[
            jax.ShapeDtypeStruct((e_total, dout), jnp.bfloat16),
            jax.ShapeDtypeStruct((8, dout), jnp.float32),
        ],
        scratch_shapes=[pltpu.VMEM((8, dout), jnp.float32)],
        **kwargs,
    )(*args)


def _bn_body(inv_e, nslices, sums_ref, gamma_ref, beta_ref, h_ref, o_ref):
    s = sums_ref[0:8, :]
    for k in range(1, nslices):
        s = s + sums_ref[8 * k:8 * (k + 1), :]
    mean = s[0:1, :] * inv_e
    ex2 = s[1:2, :] * inv_e
    var = ex2 - mean * mean
    rstd = lax.rsqrt(var + 1e-5)
    scale = gamma_ref[...] * rstd
    shift = beta_ref[...] - mean * scale
    o_ref[...] = h_ref[...].astype(jnp.float32) * scale + shift


def _bn_apply(sums, gamma, beta, h, block):
    e, dout = h.shape
    nslices = sums.shape[0] // 8
    grid = e // block
    return pl.pallas_call(
        functools.partial(_bn_body, 1.0 / e, nslices),
        grid=(grid,),
        in_specs=[
            pl.BlockSpec((8 * nslices, dout), lambda i: (0, 0)),
            pl.BlockSpec((1, dout), lambda i: (0, 0)),
            pl.BlockSpec((1, dout), lambda i: (0, 0)),
            pl.BlockSpec((block, dout), lambda i: (i, 0)),
        ],
        out_specs=pl.BlockSpec((block, dout), lambda i: (i, 0)),
        out_shape=jax.ShapeDtypeStruct((e, dout), jnp.float32),
    )(sums, gamma, beta, h)


_SLICES = [
    (0, 12800, 1600),
    (12800, 79360, 1280),
    (92160, 79360, 2560),
    (171520, 84480, 2560),
    (256000, 64000, 4000),
]


def kernel(node_feats, edge_feats, edge_index, W1, b1, W2, b2, W3, b3, gamma, beta):
    e = edge_feats.shape[0]
    de = edge_feats.shape[1]
    dout = W1.shape[1]
    w1e = W1[:de].astype(jnp.bfloat16)
    w1n = W1[de:].astype(jnp.bfloat16)
    b1r = b1.reshape(1, dout)
    w2 = W2.astype(jnp.bfloat16)
    b2r = b2.reshape(1, dout)
    w3 = W3.astype(jnp.bfloat16)
    b3r = b3.reshape(1, dout)
    src = edge_index[0]
    dst = edge_index[1]
    assert _SLICES[-1][0] + _SLICES[-1][1] == e
    gs = [_sc_gather(node_feats, src[s:s + n], dst[s:s + n])
          for s, n, _ in _SLICES]
    h = None
    sums = []
    for k, (s, n, blk) in enumerate(_SLICES):
        h, s_k = _edge_mlp_slice(
            edge_feats[s:s + n], gs[k],
            w1e, w1n, b1r, w2, b2r, w3, b3r,
            e_total=e, blk_off=s // blk, block=blk,
            h_prev=h)
        sums.append(s_k)
    out = _bn_apply(jnp.concatenate(sums, axis=0),
                    gamma.reshape(1, dout), beta.reshape(1, dout), h,
                    block=32000)
    return out
